# Initial kernel scaffold; baseline (speedup 1.0000x reference)
import jax
import jax.numpy as jnp
from jax.experimental import pallas as pl


def kernel(x, table):
    raise NotImplementedError("write your pallas kernel here")


# speedup vs baseline: 5.4919x; 5.4919x over previous
import jax
import jax.numpy as jnp
from jax import lax
from jax.experimental import pallas as pl
from jax.experimental.pallas import tpu as pltpu
from jax.experimental.pallas import tpu_sc as plsc

_MAX_LEN = 512
_D = 64
_TAB_ROWS = 2 * _MAX_LEN - 1
_TAB_WORDS = _TAB_ROWS * _D
_ROW_WORDS = _MAX_LEN * _D
_NUM_CORES = 2
_NUM_SUBCORES = 16
_NUM_WORKERS = _NUM_CORES * _NUM_SUBCORES
_WINDOW = 8


def _build_sc_kernel(batch):
    total_rows = batch * _MAX_LEN
    per_w = total_rows // _NUM_WORKERS
    mesh = plsc.VectorSubcoreMesh(core_axis_name="c", subcore_axis_name="s")

    def body(ftab_hbm, out_hbm, tab_v, sem):
        wid = lax.axis_index("c") * _NUM_SUBCORES + lax.axis_index("s")
        base = wid * per_w
        pltpu.sync_copy(ftab_hbm, tab_v)
        copies = []
        for t in range(per_w):
            r = base + t
            i = lax.rem(r, _MAX_LEN)
            off = (_MAX_LEN - 1 - i) * _D
            copies.append(
                pltpu.async_copy(tab_v.at[pl.ds(off, _ROW_WORDS)],
                                 out_hbm.at[pl.ds(r * _ROW_WORDS, _ROW_WORDS)],
                                 sem))
            if t >= _WINDOW:
                copies[t - _WINDOW].wait()
        for t in range(per_w - _WINDOW, per_w):
            copies[t].wait()

    return pl.kernel(
        body,
        out_type=jax.ShapeDtypeStruct((total_rows * _ROW_WORDS,), jnp.float32),
        mesh=mesh,
        scratch_types=[
            pltpu.VMEM((_TAB_WORDS,), jnp.float32),
            pltpu.SemaphoreType.DMA,
        ],
    )


def kernel(x, table):
    batch, seq_len = x.shape
    ftab = jnp.flip(table, axis=0).reshape(-1)
    out = _build_sc_kernel(batch)(ftab)
    return out.reshape(batch, seq_len, seq_len, _D)


# speedup vs baseline: 6.8283x; 1.2433x over previous
import jax
import jax.numpy as jnp
from jax import lax
from jax.experimental import pallas as pl
from jax.experimental.pallas import tpu as pltpu
from jax.experimental.pallas import tpu_sc as plsc

_MAX_LEN = 512
_D = 64
_TAB_ROWS = 2 * _MAX_LEN - 1
_TAB_WORDS = _TAB_ROWS * _D
_ROW_WORDS = _MAX_LEN * _D
_NUM_CORES = 2
_NUM_SUBCORES = 16
_NUM_WORKERS = _NUM_CORES * _NUM_SUBCORES
_WINDOW = 8


def _build_sc_kernel(batch):
    total_rows = batch * _MAX_LEN
    per_w = total_rows // _NUM_WORKERS
    mesh = plsc.VectorSubcoreMesh(core_axis_name="c", subcore_axis_name="s")

    def body(ftab_hbm, out_hbm, tab_v, sem):
        wid = lax.axis_index("c") * _NUM_SUBCORES + lax.axis_index("s")
        base = wid * per_w
        pltpu.sync_copy(ftab_hbm, tab_v)
        copies = []
        for t in range(per_w):
            r = base + t
            b = lax.div(r, _MAX_LEN)
            i = lax.rem(r, _MAX_LEN)
            off = _MAX_LEN - 1 - i
            copies.append(
                pltpu.async_copy(tab_v.at[pl.ds(off, _MAX_LEN), :],
                                 out_hbm.at[b, i], sem))
            if t >= _WINDOW:
                copies[t - _WINDOW].wait()
        for t in range(per_w - _WINDOW, per_w):
            copies[t].wait()

    return pl.kernel(
        body,
        out_type=jax.ShapeDtypeStruct(
            (batch, _MAX_LEN, _MAX_LEN, _D), jnp.float32),
        mesh=mesh,
        scratch_types=[
            pltpu.VMEM((_TAB_ROWS, _D), jnp.float32),
            pltpu.SemaphoreType.DMA,
        ],
    )


def kernel(x, table):
    batch, seq_len = x.shape
    del seq_len
    ftab = jnp.flip(table, axis=0)
    return _build_sc_kernel(batch)(ftab)
